# baseline (speedup 1.0000x reference)
import functools

import jax
import jax.numpy as jnp
from jax import lax
from jax.experimental import pallas as pl
from jax.experimental.pallas import tpu as pltpu
from jax.experimental.pallas import tpu_sc as plsc

B, L, D = 4096, 200, 64
DP = 128
H = B * L
NW = 32
SMALL_PER_W = B // NW
HIST_PER_W = H // NW
CHUNK = 128
OUTER = HIST_PER_W // CHUNK

_mesh = plsc.VectorSubcoreMesh(core_axis_name="c", subcore_axis_name="s")


@functools.partial(
    pl.kernel,
    mesh=_mesh,
    out_type=jax.ShapeDtypeStruct((2 * H + 3 * B, DP), jnp.float32),
    scratch_types=[
        pltpu.VMEM((HIST_PER_W,), jnp.int32),
        pltpu.VMEM((HIST_PER_W,), jnp.int32),
        pltpu.VMEM((SMALL_PER_W,), jnp.int32),
        pltpu.VMEM((CHUNK, DP), jnp.float32),
        pltpu.VMEM((CHUNK, DP), jnp.float32),
        pltpu.VMEM((CHUNK, DP), jnp.float32),
        pltpu.VMEM((CHUNK, DP), jnp.float32),
        pltpu.SemaphoreType.DMA,
        pltpu.SemaphoreType.DMA,
        pltpu.SemaphoreType.DMA,
        pltpu.SemaphoreType.DMA,
        pltpu.SemaphoreType.DMA,
        pltpu.SemaphoreType.DMA,
        pltpu.SemaphoreType.DMA,
        pltpu.SemaphoreType.DMA,
    ],
)
def _embed_all(user_i, mat_i, cat_i, mh_i, ch_i,
               user_t, mid_t, cid_t,
               all_o,
               midx_v, cidx_v, sidx_v,
               mbuf0, mbuf1, cbuf0, cbuf1,
               mg0, mg1, cg0, cg1, mw0, mw1, cw0, cw1):
    wid = lax.axis_index("s") * 2 + lax.axis_index("c")
    sbase = wid * SMALL_PER_W
    hbase = wid * HIST_PER_W

    def small(idx_hbm, table, obase):
        pltpu.sync_copy(idx_hbm.at[pl.ds(sbase, SMALL_PER_W)], sidx_v)
        pltpu.async_copy(table.at[sidx_v],
                         mbuf0.at[pl.ds(0, SMALL_PER_W)], mg0).wait()
        pltpu.sync_copy(mbuf0.at[pl.ds(0, SMALL_PER_W)],
                        all_o.at[pl.ds(obase + sbase, SMALL_PER_W)])

    small(user_i, user_t, 0)
    small(mat_i, mid_t, 2 * H + B)
    small(cat_i, cid_t, 2 * H + 2 * B)

    pltpu.sync_copy(mh_i.at[pl.ds(hbase, HIST_PER_W)], midx_v)
    pltpu.sync_copy(ch_i.at[pl.ds(hbase, HIST_PER_W)], cidx_v)

    mbufs, cbufs = (mbuf0, mbuf1), (cbuf0, cbuf1)
    mgs, cgs = (mg0, mg1), (cg0, cg1)
    mws, cws = (mw0, mw1), (cw0, cw1)

    def fire(table, idx_v, bufs, gsems, t, b):
        pltpu.async_copy(table.at[idx_v.at[pl.ds(t * CHUNK, CHUNK)]],
                         bufs[b], gsems[b])

    def drain(table, bufs, gsems, b):
        pltpu.make_async_copy(table.at[pl.ds(0, CHUNK)], bufs[b],
                              gsems[b]).wait()

    def write(obase, bufs, wsems, t, b):
        pltpu.async_copy(bufs[b],
                         all_o.at[pl.ds(obase + hbase + t * CHUNK, CHUNK)],
                         wsems[b])

    def wait_write(obase, bufs, wsems, b):
        pltpu.make_async_copy(bufs[b], all_o.at[pl.ds(obase + hbase, CHUNK)],
                              wsems[b]).wait()

    MH, CH = B, B + H

    fire(mid_t, midx_v, mbufs, mgs, 0, 0)
    fire(cid_t, cidx_v, cbufs, cgs, 0, 0)
    fire(mid_t, midx_v, mbufs, mgs, 1, 1)
    fire(cid_t, cidx_v, cbufs, cgs, 1, 1)

    @pl.loop(0, OUTER - 2, step=2)
    def _(t):
        drain(mid_t, mbufs, mgs, 0)
        write(MH, mbufs, mws, t, 0)
        drain(cid_t, cbufs, cgs, 0)
        write(CH, cbufs, cws, t, 0)
        drain(mid_t, mbufs, mgs, 1)
        write(MH, mbufs, mws, t + 1, 1)
        drain(cid_t, cbufs, cgs, 1)
        write(CH, cbufs, cws, t + 1, 1)
        wait_write(MH, mbufs, mws, 0)
        fire(mid_t, midx_v, mbufs, mgs, t + 2, 0)
        wait_write(CH, cbufs, cws, 0)
        fire(cid_t, cidx_v, cbufs, cgs, t + 2, 0)
        wait_write(MH, mbufs, mws, 1)
        fire(mid_t, midx_v, mbufs, mgs, t + 3, 1)
        wait_write(CH, cbufs, cws, 1)
        fire(cid_t, cidx_v, cbufs, cgs, t + 3, 1)

    drain(mid_t, mbufs, mgs, 0)
    write(MH, mbufs, mws, OUTER - 2, 0)
    drain(cid_t, cbufs, cgs, 0)
    write(CH, cbufs, cws, OUTER - 2, 0)
    drain(mid_t, mbufs, mgs, 1)
    write(MH, mbufs, mws, OUTER - 1, 1)
    drain(cid_t, cbufs, cgs, 1)
    write(CH, cbufs, cws, OUTER - 1, 1)
    wait_write(MH, mbufs, mws, 0)
    wait_write(CH, cbufs, cws, 0)
    wait_write(MH, mbufs, mws, 1)
    wait_write(CH, cbufs, cws, 1)


def kernel(user, material, category, material_historical, category_historical,
           material_historical_neg, category_historical_neg,
           user_table, mid_table, cid_table):
    del material_historical_neg, category_historical_neg
    ui = user.astype(jnp.int32)
    mi = material.astype(jnp.int32)
    ci = category.astype(jnp.int32)
    mh = material_historical.astype(jnp.int32).reshape(H)
    ch = category_historical.astype(jnp.int32).reshape(H)
    pad = ((0, 0), (0, DP - D))
    user_tp = jnp.pad(user_table, pad)
    mid_tp = jnp.pad(mid_table, pad)
    cid_tp = jnp.pad(cid_table, pad)
    all_e = _embed_all(ui, mi, ci, mh, ch, user_tp, mid_tp, cid_tp)
    all_d = all_e[:, :D]
    return (all_d[:B],
            all_d[B:B + H].reshape(B, L, D),
            all_d[B + H:B + 2 * H].reshape(B, L, D),
            all_d[B + 2 * H:2 * B + 2 * H],
            all_d[2 * B + 2 * H:])


# speedup vs baseline: 1.3249x; 1.3249x over previous
import functools

import jax
import jax.numpy as jnp
from jax import lax
from jax.experimental import pallas as pl
from jax.experimental.pallas import tpu as pltpu
from jax.experimental.pallas import tpu_sc as plsc

B, L, D = 4096, 200, 64
DP = 128
H = B * L
NW = 32
SMALL_PER_W = B // NW
HIST_PER_W = H // NW
CHUNK = 128
OUTER = HIST_PER_W // CHUNK

_mesh = plsc.VectorSubcoreMesh(core_axis_name="c", subcore_axis_name="s")


@functools.partial(
    pl.kernel,
    mesh=_mesh,
    out_type=[
        jax.ShapeDtypeStruct((B, DP), jnp.float32),
        jax.ShapeDtypeStruct((H, DP), jnp.float32),
        jax.ShapeDtypeStruct((H, DP), jnp.float32),
        jax.ShapeDtypeStruct((B, DP), jnp.float32),
        jax.ShapeDtypeStruct((B, DP), jnp.float32),
    ],
    scratch_types=[
        pltpu.VMEM((HIST_PER_W,), jnp.int32),
        pltpu.VMEM((HIST_PER_W,), jnp.int32),
        pltpu.VMEM((SMALL_PER_W,), jnp.int32),
        pltpu.VMEM((CHUNK, DP), jnp.float32),
        pltpu.VMEM((CHUNK, DP), jnp.float32),
        pltpu.VMEM((CHUNK, DP), jnp.float32),
        pltpu.VMEM((CHUNK, DP), jnp.float32),
        pltpu.SemaphoreType.DMA,
        pltpu.SemaphoreType.DMA,
        pltpu.SemaphoreType.DMA,
        pltpu.SemaphoreType.DMA,
        pltpu.SemaphoreType.DMA,
        pltpu.SemaphoreType.DMA,
        pltpu.SemaphoreType.DMA,
        pltpu.SemaphoreType.DMA,
    ],
)
def _embed_all(user_i, mat_i, cat_i, mh_i, ch_i,
               user_t, mid_t, cid_t,
               user_o, mh_o, ch_o, mat_o, cat_o,
               midx_v, cidx_v, sidx_v,
               mbuf0, mbuf1, cbuf0, cbuf1,
               mg0, mg1, cg0, cg1, mw0, mw1, cw0, cw1):
    wid = lax.axis_index("s") * 2 + lax.axis_index("c")
    sbase = wid * SMALL_PER_W
    hbase = wid * HIST_PER_W

    def small(idx_hbm, table, out):
        pltpu.sync_copy(idx_hbm.at[pl.ds(sbase, SMALL_PER_W)], sidx_v)
        pltpu.async_copy(table.at[sidx_v],
                         mbuf0.at[pl.ds(0, SMALL_PER_W)], mg0).wait()
        pltpu.sync_copy(mbuf0.at[pl.ds(0, SMALL_PER_W)],
                        out.at[pl.ds(sbase, SMALL_PER_W)])

    small(user_i, user_t, user_o)
    small(mat_i, mid_t, mat_o)
    small(cat_i, cid_t, cat_o)

    pltpu.sync_copy(mh_i.at[pl.ds(hbase, HIST_PER_W)], midx_v)
    pltpu.sync_copy(ch_i.at[pl.ds(hbase, HIST_PER_W)], cidx_v)

    mbufs, cbufs = (mbuf0, mbuf1), (cbuf0, cbuf1)
    mgs, cgs = (mg0, mg1), (cg0, cg1)
    mws, cws = (mw0, mw1), (cw0, cw1)

    def fire(table, idx_v, bufs, gsems, t, b):
        pltpu.async_copy(table.at[idx_v.at[pl.ds(t * CHUNK, CHUNK)]],
                         bufs[b], gsems[b])

    def drain(table, bufs, gsems, b):
        pltpu.make_async_copy(table.at[pl.ds(0, CHUNK)], bufs[b],
                              gsems[b]).wait()

    def write(out, bufs, wsems, t, b):
        pltpu.async_copy(bufs[b], out.at[pl.ds(hbase + t * CHUNK, CHUNK)],
                         wsems[b])

    def wait_write(out, bufs, wsems, b):
        pltpu.make_async_copy(bufs[b], out.at[pl.ds(hbase, CHUNK)],
                              wsems[b]).wait()

    fire(mid_t, midx_v, mbufs, mgs, 0, 0)
    fire(cid_t, cidx_v, cbufs, cgs, 0, 0)
    fire(mid_t, midx_v, mbufs, mgs, 1, 1)
    fire(cid_t, cidx_v, cbufs, cgs, 1, 1)

    @pl.loop(0, OUTER - 2, step=2)
    def _(t):
        drain(mid_t, mbufs, mgs, 0)
        write(mh_o, mbufs, mws, t, 0)
        drain(cid_t, cbufs, cgs, 0)
        write(ch_o, cbufs, cws, t, 0)
        drain(mid_t, mbufs, mgs, 1)
        write(mh_o, mbufs, mws, t + 1, 1)
        drain(cid_t, cbufs, cgs, 1)
        write(ch_o, cbufs, cws, t + 1, 1)
        wait_write(mh_o, mbufs, mws, 0)
        fire(mid_t, midx_v, mbufs, mgs, t + 2, 0)
        wait_write(ch_o, cbufs, cws, 0)
        fire(cid_t, cidx_v, cbufs, cgs, t + 2, 0)
        wait_write(mh_o, mbufs, mws, 1)
        fire(mid_t, midx_v, mbufs, mgs, t + 3, 1)
        wait_write(ch_o, cbufs, cws, 1)
        fire(cid_t, cidx_v, cbufs, cgs, t + 3, 1)

    drain(mid_t, mbufs, mgs, 0)
    write(mh_o, mbufs, mws, OUTER - 2, 0)
    drain(cid_t, cbufs, cgs, 0)
    write(ch_o, cbufs, cws, OUTER - 2, 0)
    drain(mid_t, mbufs, mgs, 1)
    write(mh_o, mbufs, mws, OUTER - 1, 1)
    drain(cid_t, cbufs, cgs, 1)
    write(ch_o, cbufs, cws, OUTER - 1, 1)
    wait_write(mh_o, mbufs, mws, 0)
    wait_write(ch_o, cbufs, cws, 0)
    wait_write(mh_o, mbufs, mws, 1)
    wait_write(ch_o, cbufs, cws, 1)


def kernel(user, material, category, material_historical, category_historical,
           material_historical_neg, category_historical_neg,
           user_table, mid_table, cid_table):
    del material_historical_neg, category_historical_neg
    ui = user.astype(jnp.int32)
    mi = material.astype(jnp.int32)
    ci = category.astype(jnp.int32)
    mh = material_historical.astype(jnp.int32).reshape(H)
    ch = category_historical.astype(jnp.int32).reshape(H)
    pad = ((0, 0), (0, DP - D))
    user_tp = jnp.pad(user_table, pad)
    mid_tp = jnp.pad(mid_table, pad)
    cid_tp = jnp.pad(cid_table, pad)
    user_e, mh_e, ch_e, mat_e, cat_e = _embed_all(
        ui, mi, ci, mh, ch, user_tp, mid_tp, cid_tp)
    return (user_e[:, :D],
            mh_e[:, :D].reshape(B, L, D),
            ch_e[:, :D].reshape(B, L, D),
            mat_e[:, :D],
            cat_e[:, :D])
